# baseline (speedup 1.0000x reference)
import jax
import jax.numpy as jnp
from jax import lax
from jax.experimental import pallas as pl
from jax.experimental.pallas import tpu as pltpu

_NUM_CLASSES = 80
_STRIDE = 32
_A = 5
_THRESH = 0.6
_OBJECT_SCALE = 5.0
_NOOBJECT_SCALE = 1.0


def _body(x0, x1, x2, x3, x4, t_ref, a_ref, o_ref):
    f32 = jnp.float32
    N = t_ref.shape[1]
    xs = (x0, x1, x2, x3, x4)

    r = lax.broadcasted_iota(jnp.int32, (8, 128), 0)
    l = lax.broadcasted_iota(jnp.int32, (8, 128), 1)
    gx = (l % 32).astype(f32)
    gy = ((l // 32) * 8 + r).astype(f32)

    gt = []
    for j in range(N):
        gcls = t_ref[0, j, 0]
        gcx = t_ref[0, j, 1]
        gcy = t_ref[0, j, 2]
        gw = t_ref[0, j, 3]
        gh = t_ref[0, j, 4]
        g1x = gcx - gw / 2
        g1y = gcy - gh / 2
        g2x = gcx + gw / 2
        g2y = gcy + gh / 2
        garea = (g2x - g1x) * (g2y - g1y)
        gt.append((gcls, gcx, gcy, gw, gh, g1x, g1y, g2x, g2y, garea))

    acc = jnp.zeros((8, 128), f32)
    for a in range(_A):
        x_ref = xs[a]

        def ld(ch):
            return jnp.concatenate(
                [x_ref[0, ch, 8 * b : 8 * b + 8, :] for b in range(4)], axis=1
            )

        tx = ld(0)
        ty = ld(1)
        tw = ld(2)
        th = ld(3)
        conf = ld(4)
        aw = a_ref[a, 0]
        ah = a_ref[a, 1]

        px = (1.0 / (1.0 + jnp.exp(-tx)) + gx) * float(_STRIDE)
        py = (1.0 / (1.0 + jnp.exp(-ty)) + gy) * float(_STRIDE)
        pw = jnp.exp(tw) * aw
        ph = jnp.exp(th) * ah
        p1x = px - pw / 2
        p1y = py - ph / 2
        p2x = px + pw / 2
        p2y = py + ph / 2
        parea = (p2x - p1x) * (p2y - p1y)

        z = jnp.zeros((8, 128), f32)
        best_iou = jnp.full((8, 128), -1.0, f32)
        bx, by, bw, bh, bcls = z, z, z, z, z
        for j in range(N):
            gcls, gcx, gcy, gw, gh, g1x, g1y, g2x, g2y, garea = gt[j]
            x1 = jnp.maximum(g1x, p1x)
            y1 = jnp.maximum(g1y, p1y)
            x2 = jnp.minimum(g2x, p2x)
            y2 = jnp.minimum(g2y, p2y)
            inter = jnp.maximum(x2 - x1, 0.0) * jnp.maximum(y2 - y1, 0.0)
            union = garea + parea - inter + 1e-6
            iou = inter / union
            upd = iou > best_iou
            best_iou = jnp.maximum(iou, best_iou)
            bx = jnp.where(upd, gcx, bx)
            by = jnp.where(upd, gcy, by)
            bw = jnp.where(upd, gw, bw)
            bh = jnp.where(upd, gh, bh)
            bcls = jnp.where(upd, gcls, bcls)

        mask = best_iou > _THRESH
        cm = jnp.where(mask, 1.0, 0.0)
        scale = jnp.where(mask, _OBJECT_SCALE, _NOOBJECT_SCALE)

        dx = tx * cm - bx * cm
        dy = ty * cm - by * cm
        dw = tw * cm - bw * cm
        dh = th * cm - bh * cm
        coord_l = dx * dx + dy * dy + dw * dw + dh * dh

        dc = conf * scale - cm * scale
        conf_l = dc * dc

        m = ld(5)
        for ci in range(1, _NUM_CLASSES):
            m = jnp.maximum(m, ld(5 + ci))
        ssum = jnp.zeros((8, 128), f32)
        picked = jnp.zeros((8, 128), f32)
        for ci in range(_NUM_CLASSES):
            v = ld(5 + ci)
            ssum = ssum + jnp.exp(v - m)
            picked = jnp.where(bcls == float(ci), v, picked)
        ce = jnp.log(ssum) - (picked - m)
        cls_l = cm * ce

        acc = acc + coord_l + conf_l + cls_l

    o_ref[0, 0, 0] = jnp.sum(acc)


def kernel(output, target, anchors):
    B = output.shape[0]
    C85 = 5 + _NUM_CLASSES

    def mk_spec(a):
        return pl.BlockSpec((1, C85, 32, 32), lambda b, _a=a: (b, _a, 0, 0))

    partial = pl.pallas_call(
        _body,
        grid=(B,),
        in_specs=[mk_spec(a) for a in range(_A)]
        + [
            pl.BlockSpec((1, target.shape[1], 5), lambda b: (b, 0, 0), memory_space=pltpu.SMEM),
            pl.BlockSpec((_A, 2), lambda b: (0, 0), memory_space=pltpu.SMEM),
        ],
        out_specs=pl.BlockSpec((1, 1, 1), lambda b: (b, 0, 0), memory_space=pltpu.SMEM),
        out_shape=jax.ShapeDtypeStruct((B, 1, 1), jnp.float32),
    )(output, output, output, output, output, target, anchors)
    return jnp.sum(partial)
